# Initial kernel scaffold; baseline (speedup 1.0000x reference)
import jax
import jax.numpy as jnp
from jax.experimental import pallas as pl


def kernel(x, timestamp, tok_table, time_table, gamma, beta):
    raise NotImplementedError("write your pallas kernel here")


# speedup vs baseline: 1.9927x; 1.9927x over previous
import functools
import numpy as np
import jax
import jax.numpy as jnp
from jax import lax
from jax.experimental import pallas as pl
from jax.experimental.pallas import tpu as pltpu
from jax.experimental.pallas import tpu_sc as plsc

EPS = 1e-5
NW = 32
CHUNK = 100


def _make_pe(max_len, d):
    position = np.arange(max_len, dtype=np.float32)[:, None]
    div_term = np.exp(np.arange(0, d, 2, dtype=np.float32) * -(np.log(10000.0) / d))
    pe = np.zeros((max_len, d), dtype=np.float32)
    pe[:, 0::2] = np.sin(position * div_term)
    pe[:, 1::2] = np.cos(position * div_term)
    return pe


def _build(B, L, D):
    assert B % NW == 0 and L % CHUNK == 0 and D % 16 == 0
    seq_per_tile = B // NW
    nch = L // CHUNK
    nk = D // 16

    @functools.partial(
        pl.kernel,
        mesh=plsc.VectorSubcoreMesh(core_axis_name="c", subcore_axis_name="s"),
        out_type=jax.ShapeDtypeStruct((B, L, D), jnp.float32),
        compiler_params=pltpu.CompilerParams(use_tc_tiling_on_sc=False),
        scratch_types=[
            pltpu.VMEM((nch, CHUNK), jnp.int32),
            pltpu.VMEM((nch, CHUNK), jnp.int32),
            pltpu.VMEM((L, D), jnp.float32),
            pltpu.VMEM((L, D), jnp.float32),
            pltpu.VMEM((L, D), jnp.float32),
            pltpu.VMEM((L, D), jnp.float32),
            pltpu.VMEM((D,), jnp.float32),
            pltpu.VMEM((D,), jnp.float32),
            pltpu.SemaphoreType.DMA,
        ],
    )
    def _k(x_hbm, ts_hbm, tok_hbm, tim_hbm, pe_hbm, g_hbm, b_hbm, out_hbm,
           xidx, tidx, tokb, timb, peb, outb, gb, bb, sem):
        wid = lax.axis_index("s") * 2 + lax.axis_index("c")
        pltpu.sync_copy(pe_hbm, peb)
        pltpu.sync_copy(g_hbm, gb)
        pltpu.sync_copy(b_hbm, bb)
        gvs = [gb[pl.ds(16 * k, 16)] for k in range(nk)]
        bvs = [bb[pl.ds(16 * k, 16)] for k in range(nk)]
        base = wid * seq_per_tile
        lane = lax.broadcasted_iota(jnp.int32, (16,), 0)
        perms = [(lane + sh) & 15 for sh in (8, 4, 2, 1)]

        dnums = lax.GatherDimensionNumbers(
            offset_dims=(), collapsed_slice_dims=(0,), start_index_map=(0,))

        def shuffle(v, p):
            return lax.gather(v, p[:, None], dnums, (1,),
                              mode=lax.GatherScatterMode.PROMISE_IN_BOUNDS)

        def lanesum(v):
            for p in perms:
                v = v + shuffle(v, p)
            return v

        def seq_body(t, carry):
            seq = base + t
            pltpu.sync_copy(x_hbm.at[seq], xidx)
            pltpu.sync_copy(ts_hbm.at[seq], tidx)
            cs = []
            for j in range(nch):
                dst = pl.ds(j * CHUNK, CHUNK)
                cs.append(pltpu.async_copy(tok_hbm.at[xidx.at[j]], tokb.at[dst], sem))
                cs.append(pltpu.async_copy(tim_hbm.at[tidx.at[j]], timb.at[dst], sem))
            for c in cs:
                c.wait()

            def row_body(r, c2):
                e = []
                for k in range(nk):
                    sl = pl.ds(16 * k, 16)
                    e.append(tokb[r, sl] + timb[r, sl] + peb[r, sl])
                s = (e[0] + e[1]) + (e[2] + e[3])
                q = (e[0] * e[0] + e[1] * e[1]) + (e[2] * e[2] + e[3] * e[3])
                inv_d = jnp.float32(1.0 / D)
                mu = lanesum(s) * inv_d
                ms = lanesum(q) * inv_d
                var = ms - mu * mu
                xx = var + jnp.float32(EPS)
                i = lax.bitcast_convert_type(xx, jnp.int32)
                i = jnp.int32(0x5F3759DF) - lax.shift_right_arithmetic(i, 1)
                y = lax.bitcast_convert_type(i, jnp.float32)
                for _ in range(3):
                    y = y * (jnp.float32(1.5) - jnp.float32(0.5) * xx * y * y)
                for k in range(nk):
                    sl = pl.ds(16 * k, 16)
                    outb[r, sl] = (e[k] - mu) * y * gvs[k] + bvs[k]
                return c2

            lax.fori_loop(0, L, row_body, 0)
            pltpu.sync_copy(outb, out_hbm.at[seq])
            return carry

        lax.fori_loop(0, seq_per_tile, seq_body, 0)

    return _k


def kernel(x, timestamp, tok_table, time_table, gamma, beta):
    B, L = x.shape
    D = tok_table.shape[1]
    pe = jnp.asarray(_make_pe(L, D))
    nch = L // CHUNK
    x3 = x.reshape(B, nch, CHUNK)
    ts3 = timestamp.reshape(B, nch, CHUNK)
    return _build(B, L, D)(x3, ts3, tok_table, time_table, pe, gamma, beta)
